# baseline (speedup 1.0000x reference)
import functools

import jax
import jax.numpy as jnp
from jax import lax
from jax.experimental import pallas as pl
from jax.experimental.pallas import tpu as pltpu
from jax.experimental.pallas import tpu_sc as plsc

N = 10000
E = 160000
G = 256
NPAD = 10240
EPAD = 163840
NC, NS = 2, 16
NW = NC * NS
EW = EPAD // NW
WN = 64
NWIN = EW // WN
RW = NPAD // NS
BM = 1024


def _dot3(s, w):
    dn = (((1,), (0,)), ((), ()))
    sh = s.astype(jnp.bfloat16)
    sl = (s - sh.astype(jnp.float32)).astype(jnp.bfloat16)
    wh = w.astype(jnp.bfloat16)
    wl = (w - wh.astype(jnp.float32)).astype(jnp.bfloat16)
    out = lax.dot_general(sh, wh, dn, preferred_element_type=jnp.float32)
    out = out + lax.dot_general(sh, wl, dn, preferred_element_type=jnp.float32)
    out = out + lax.dot_general(sl, wh, dn, preferred_element_type=jnp.float32)
    return out


def _mesh():
    return plsc.VectorSubcoreMesh(core_axis_name="c", subcore_axis_name="s")


def _sc_degree(dst3, ones_h, zrow_h):
    @functools.partial(
        pl.kernel,
        out_type=jax.ShapeDtypeStruct((NC, NPAD), jnp.float32),
        mesh=_mesh(),
        scratch_types=[
            pltpu.VMEM((NWIN, WN), jnp.int32),
            pltpu.VMEM((WN,), jnp.float32),
            pltpu.VMEM((128,), jnp.float32),
            pltpu.VMEM_SHARED((NPAD,), jnp.float32),
        ],
    )
    def k(dst_h, ones_hbm, z_hbm, deg_out, idx_v, ones_v, z_v, acc):
        ci = lax.axis_index("c")
        si = lax.axis_index("s")
        w = ci * NS + si
        pltpu.sync_copy(dst_h.at[w], idx_v)
        pltpu.sync_copy(ones_hbm, ones_v)
        pltpu.sync_copy(z_hbm, z_v)

        @pl.loop(0, RW // 128)
        def _zero(j):
            pltpu.sync_copy(z_v, acc.at[pl.ds(si * RW + j * 128, 128)])

        plsc.subcore_barrier()

        @pl.loop(0, NWIN)
        def _scat(j):
            pltpu.sync_copy(ones_v, acc.at[idx_v.at[j]], add=True)

        plsc.subcore_barrier()

        @pl.loop(0, RW // 128)
        def _dump(j):
            off = si * RW + j * 128
            pltpu.sync_copy(acc.at[pl.ds(off, 128)], deg_out.at[ci, pl.ds(off, 128)])

    return k(dst3, ones_h, zrow_h)


def _sc_agg(us, src3, dst3, zt_h):
    C = len(us)
    outs = tuple(jax.ShapeDtypeStruct((NC, NPAD, 128), jnp.float32) for _ in range(C))

    @functools.partial(
        pl.kernel,
        out_type=outs,
        mesh=_mesh(),
        scratch_types=[
            pltpu.VMEM((NWIN, WN), jnp.int32),
            pltpu.VMEM((NWIN, WN), jnp.int32),
            pltpu.VMEM((WN, 128), jnp.float32),
            pltpu.VMEM((WN, 128), jnp.float32),
            pltpu.VMEM((WN, 128), jnp.float32),
            pltpu.VMEM_SHARED((NPAD, 128), jnp.float32),
            pltpu.SemaphoreType.DMA,
            pltpu.SemaphoreType.DMA,
            pltpu.SemaphoreType.DMA,
            pltpu.SemaphoreType.DMA,
            pltpu.SemaphoreType.DMA,
            pltpu.SemaphoreType.DMA,
        ],
    )
    def k(*refs):
        u_refs = refs[:C]
        src_h, dst_h, zt = refs[C], refs[C + 1], refs[C + 2]
        out_refs = refs[C + 3:2 * C + 3]
        sc = refs[2 * C + 3:]
        si_v, di_v = sc[0], sc[1]
        bufs = sc[2:5]
        acc = sc[5]
        gs = sc[6:9]
        ss = sc[9:12]
        ci = lax.axis_index("c")
        si = lax.axis_index("s")
        w = ci * NS + si
        pltpu.sync_copy(src_h.at[w], si_v)
        pltpu.sync_copy(dst_h.at[w], di_v)
        for c in range(C):
            u = u_refs[c]
            out = out_refs[c]

            pltpu.sync_copy(zt, bufs[2])
            for p in range(RW // WN):
                pltpu.async_copy(
                    bufs[2], acc.at[pl.ds(si * RW + p * WN, WN)], gs[1])
            for p in range(RW // WN):
                pltpu.make_async_copy(
                    bufs[2], acc.at[pl.ds(0, WN)], gs[1]).wait()

            plsc.subcore_barrier()

            for t in range(2):
                pltpu.async_copy(u.at[si_v.at[t]], bufs[t], gs[t])

            @pl.loop(0, NWIN - 2, step=3)
            def _edges(j, u=u):
                for t in range(3):
                    tn = (t + 2) % 3
                    pltpu.make_async_copy(u.at[pl.ds(0, WN)], bufs[t], gs[t]).wait()
                    pltpu.async_copy(bufs[t], acc.at[di_v.at[j + t]], ss[t], add=True)

                    @pl.when(j + t > 0)
                    def _ws(tn=tn):
                        pltpu.make_async_copy(
                            bufs[tn], acc.at[pl.ds(0, WN)], ss[tn]).wait()

                    pltpu.async_copy(u.at[si_v.at[j + t + 2]], bufs[tn], gs[tn])

            pltpu.make_async_copy(u.at[pl.ds(0, WN)], bufs[0], gs[0]).wait()
            pltpu.async_copy(bufs[0], acc.at[di_v.at[NWIN - 2]], ss[0], add=True)
            pltpu.make_async_copy(bufs[2], acc.at[pl.ds(0, WN)], ss[2]).wait()
            pltpu.make_async_copy(u.at[pl.ds(0, WN)], bufs[1], gs[1]).wait()
            pltpu.async_copy(bufs[1], acc.at[di_v.at[NWIN - 1]], ss[1], add=True)
            pltpu.make_async_copy(bufs[0], acc.at[pl.ds(0, WN)], ss[0]).wait()
            pltpu.make_async_copy(bufs[1], acc.at[pl.ds(0, WN)], ss[1]).wait()
            plsc.subcore_barrier()

            for p in range(RW // 128):
                off = si * RW + p * 128
                pltpu.async_copy(
                    acc.at[pl.ds(off, 128)], out.at[ci, pl.ds(off, 128)], gs[0])
            for p in range(RW // 128):
                pltpu.make_async_copy(
                    acc.at[pl.ds(0, 128)], out.at[ci, pl.ds(0, 128)], gs[0]).wait()

            plsc.subcore_barrier()

    return k(*us, src3, dst3, zt_h)


def _tc_dinv(deg2):
    def body(d_ref, dv_ref):
        d = d_ref[...]
        t = d[0] + d[1] + 1.0
        r = lax.rsqrt(jnp.maximum(t, 1.0))
        dv_ref[...] = jnp.broadcast_to(r[:, None], (NPAD, 128))

    return pl.pallas_call(
        body,
        grid=(1,),
        in_specs=[pl.BlockSpec((NC, NPAD), lambda i: (0, 0))],
        out_specs=pl.BlockSpec((NPAD, 128), lambda i: (0, 0)),
        out_shape=jax.ShapeDtypeStruct((NPAD, 128), jnp.float32),
    )(deg2)


def _tc_scale_chunks(xp, dinv, C):
    def body(x_ref, dv_ref, *outs):
        xx = x_ref[...]
        dv = dv_ref[...]
        for c in range(C):
            outs[c][...] = xx[:, c * 128:(c + 1) * 128] * dv

    return pl.pallas_call(
        body,
        grid=(NPAD // BM,),
        in_specs=[
            pl.BlockSpec((BM, C * 128), lambda m: (m, 0)),
            pl.BlockSpec((BM, 128), lambda m: (m, 0)),
        ],
        out_specs=tuple(pl.BlockSpec((BM, 128), lambda m: (m, 0)) for _ in range(C)),
        out_shape=tuple(jax.ShapeDtypeStruct((NPAD, 128), jnp.float32) for _ in range(C)),
    )(xp, dinv)


def _tc_fuse_mm(aggs, us, dinv, W, brow, dout):
    C = len(us)

    def body(*refs):
        a = refs[:C]
        u = refs[C:2 * C]
        dv = refs[2 * C][...]
        w_all = refs[2 * C + 1][...]
        b_ = refs[2 * C + 2][...]
        o_ref = refs[2 * C + 3]
        st_ref = refs[2 * C + 4]
        m = pl.program_id(0)
        s = jnp.concatenate(
            [(a[c][...][0] + a[c][...][1] + u[c][...]) * dv for c in range(C)],
            axis=1)
        o = jnp.broadcast_to(b_, (BM, dout)) + _dot3(s, w_all)
        o_ref[...] = o
        rows = lax.broadcasted_iota(jnp.int32, (BM, 1), 0) + m * BM
        om = jnp.where(rows < N, o, 0.0)
        part = jnp.concatenate(
            [jnp.sum(om, axis=0)[None, :], jnp.sum(om * om, axis=0)[None, :],
             jnp.zeros((6, dout), jnp.float32)], axis=0)

        @pl.when(m == 0)
        def _init():
            st_ref[...] = part

        @pl.when(m > 0)
        def _acc():
            st_ref[...] += part

    din = C * 128
    return pl.pallas_call(
        body,
        grid=(NPAD // BM,),
        in_specs=(
            [pl.BlockSpec((NC, BM, 128), lambda m: (0, m, 0)) for _ in range(C)]
            + [pl.BlockSpec((BM, 128), lambda m: (m, 0)) for _ in range(C)]
            + [
                pl.BlockSpec((BM, 128), lambda m: (m, 0)),
                pl.BlockSpec((din, dout), lambda m: (0, 0)),
                pl.BlockSpec((1, dout), lambda m: (0, 0)),
            ]
        ),
        out_specs=(
            pl.BlockSpec((BM, dout), lambda m: (m, 0)),
            pl.BlockSpec((8, dout), lambda m: (0, 0)),
        ),
        out_shape=(
            jax.ShapeDtypeStruct((NPAD, dout), jnp.float32),
            jax.ShapeDtypeStruct((8, dout), jnp.float32),
        ),
    )(*aggs, *us, dinv, W, brow)


def _tc_mm(us, W, dout):
    C = len(us)

    def body(*refs):
        u = refs[:C]
        w_all = refs[C][...]
        o_ref = refs[C + 1]
        s = jnp.concatenate([u[c][...] for c in range(C)], axis=1)
        o_ref[...] = _dot3(s, w_all)

    din = C * 128
    return pl.pallas_call(
        body,
        grid=(NPAD // BM,),
        in_specs=(
            [pl.BlockSpec((BM, 128), lambda m: (m, 0)) for _ in range(C)]
            + [pl.BlockSpec((din, dout), lambda m: (0, 0))]
        ),
        out_specs=pl.BlockSpec((BM, dout), lambda m: (m, 0)),
        out_shape=jax.ShapeDtypeStruct((NPAD, dout), jnp.float32),
    )(*us, W)


def _tc_bnapply(o, st, grow, berow, dinv, C, d):
    def body(o_ref, st_ref, g_ref, be_ref, dv_ref, *outs):
        st_ = st_ref[...]
        mu = st_[0] / N
        var = st_[1] / N - mu * mu
        scale = lax.rsqrt(var + 1e-5) * g_ref[...][0]
        xb = (o_ref[...] - mu) * scale + be_ref[...][0]
        xb = jnp.maximum(xb, 0.0)
        dv = dv_ref[...]
        for c in range(C):
            outs[c][...] = xb[:, c * 128:(c + 1) * 128] * dv

    return pl.pallas_call(
        body,
        grid=(NPAD // BM,),
        in_specs=[
            pl.BlockSpec((BM, d), lambda m: (m, 0)),
            pl.BlockSpec((8, d), lambda m: (0, 0)),
            pl.BlockSpec((1, d), lambda m: (0, 0)),
            pl.BlockSpec((1, d), lambda m: (0, 0)),
            pl.BlockSpec((BM, 128), lambda m: (m, 0)),
        ],
        out_specs=tuple(pl.BlockSpec((BM, 128), lambda m: (m, 0)) for _ in range(C)),
        out_shape=tuple(jax.ShapeDtypeStruct((NPAD, 128), jnp.float32) for _ in range(C)),
    )(o, st, grow, berow, dinv)


def _tc_o3_stats(a3, v, dinv, b3row):
    def body(a_ref, v_ref, dv_ref, b_ref, o_ref, st_ref):
        a = a_ref[...]
        o = (a[0] + a[1] + v_ref[...]) * dv_ref[...] + b_ref[...][0]
        o_ref[...] = o
        s1 = jnp.sum(o, axis=0)
        s2 = jnp.sum(o * o, axis=0)
        st_ref[...] = jnp.concatenate(
            [s1[None, :], s2[None, :], jnp.zeros((6, 128), jnp.float32)], axis=0)

    return pl.pallas_call(
        body,
        grid=(1,),
        in_specs=[
            pl.BlockSpec((NC, N, 128), lambda i: (0, 0, 0)),
            pl.BlockSpec((N, 128), lambda i: (0, 0)),
            pl.BlockSpec((N, 128), lambda i: (0, 0)),
            pl.BlockSpec((1, 128), lambda i: (0, 0)),
        ],
        out_specs=(
            pl.BlockSpec((N, 128), lambda i: (0, 0)),
            pl.BlockSpec((8, 128), lambda i: (0, 0)),
        ),
        out_shape=(
            jax.ShapeDtypeStruct((N, 128), jnp.float32),
            jax.ShapeDtypeStruct((8, 128), jnp.float32),
        ),
    )(a3, v, dinv, b3row)


def _tc_pool(o3, st3, g3row, be3row, batch_row):
    def body(o_ref, st_ref, g_ref, be_ref, b_ref, out_ref):
        st_ = st_ref[...]
        mu = st_[0] / N
        var = st_[1] / N - mu * mu
        scale = lax.rsqrt(var + 1e-5) * g_ref[...][0]
        xb = (o_ref[...] - mu) * scale + be_ref[...][0]
        seg = b_ref[...]
        gid = lax.broadcasted_iota(jnp.int32, (G, 1), 0)
        oht = (seg == gid).astype(jnp.float32)
        sums = _dot3(oht, xb)
        cnt = jnp.sum(oht, axis=1)[:, None]
        pooled = sums / jnp.maximum(cnt, 1.0)
        nrm = jnp.sqrt(jnp.sum(pooled * pooled, axis=1, keepdims=True))
        out_ref[...] = pooled / jnp.maximum(nrm, 1e-12)

    return pl.pallas_call(
        body,
        grid=(1,),
        in_specs=[
            pl.BlockSpec((N, 128), lambda i: (0, 0)),
            pl.BlockSpec((8, 128), lambda i: (0, 0)),
            pl.BlockSpec((1, 128), lambda i: (0, 0)),
            pl.BlockSpec((1, 128), lambda i: (0, 0)),
            pl.BlockSpec((1, N), lambda i: (0, 0)),
        ],
        out_specs=pl.BlockSpec((G, 128), lambda i: (0, 0)),
        out_shape=jax.ShapeDtypeStruct((G, 128), jnp.float32),
    )(o3, st3, g3row, be3row, batch_row)


def kernel(x, edge_index, batch, W1, b1, g1, be1, W2, b2, g2, be2, W3, b3, g3, be3):
    f32 = jnp.float32
    src = edge_index[0]
    dst = edge_index[1]
    pad = (N + jnp.arange(EPAD - E, dtype=jnp.int32) % (NPAD - N)).astype(jnp.int32)
    src3 = jnp.concatenate([src, pad]).reshape(NW, NWIN, WN)
    dst3 = jnp.concatenate([dst, pad]).reshape(NW, NWIN, WN)
    xp = jnp.pad(x, ((0, NPAD - N), (0, 0)))
    ones_h = jnp.ones((WN,), f32)
    zrow_h = jnp.zeros((128,), f32)
    zt_h = jnp.zeros((WN, 128), f32)
    b1r, b2r, b3r = b1.reshape(1, -1), b2.reshape(1, -1), b3.reshape(1, -1)
    g1r, g2r, g3r = g1.reshape(1, -1), g2.reshape(1, -1), g3.reshape(1, -1)
    be1r, be2r, be3r = be1.reshape(1, -1), be2.reshape(1, -1), be3.reshape(1, -1)
    batch_row = batch.reshape(1, N)

    deg2 = _sc_degree(dst3, ones_h, zrow_h)
    dinv = _tc_dinv(deg2)

    u1 = _tc_scale_chunks(xp, dinv, 2)
    a1 = _sc_agg(u1, src3, dst3, zt_h)
    o1, st1 = _tc_fuse_mm(a1, u1, dinv, W1, b1r, 512)
    u2 = _tc_bnapply(o1, st1, g1r, be1r, dinv, 4, 512)

    a2 = _sc_agg(u2, src3, dst3, zt_h)
    o2, st2 = _tc_fuse_mm(a2, u2, dinv, W2, b2r, 512)
    q = _tc_bnapply(o2, st2, g2r, be2r, dinv, 4, 512)

    v = _tc_mm(q, W3, 128)
    a3 = _sc_agg((v,), src3, dst3, zt_h)
    o3, st3 = _tc_o3_stats(a3[0], v, dinv, b3r)
    return _tc_pool(o3, st3, g3r, be3r, batch_row)


# speedup vs baseline: 1.0176x; 1.0176x over previous
import functools

import jax
import jax.numpy as jnp
from jax import lax
from jax.experimental import pallas as pl
from jax.experimental.pallas import tpu as pltpu
from jax.experimental.pallas import tpu_sc as plsc

N = 10000
E = 160000
G = 256
NPAD = 10240
EPAD = 163840
NC, NS = 2, 16
NW = NC * NS
EW = EPAD // NW
WN = 64
NWIN = EW // WN
RW = NPAD // NS
BM = 1024


def _dot3(s, w):
    dn = (((1,), (0,)), ((), ()))
    sh = s.astype(jnp.bfloat16)
    sl = (s - sh.astype(jnp.float32)).astype(jnp.bfloat16)
    wh = w.astype(jnp.bfloat16)
    wl = (w - wh.astype(jnp.float32)).astype(jnp.bfloat16)
    out = lax.dot_general(sh, wh, dn, preferred_element_type=jnp.float32)
    out = out + lax.dot_general(sh, wl, dn, preferred_element_type=jnp.float32)
    out = out + lax.dot_general(sl, wh, dn, preferred_element_type=jnp.float32)
    return out


def _mesh():
    return plsc.VectorSubcoreMesh(core_axis_name="c", subcore_axis_name="s")


def _sc_degree(dst3, ones_h, zrow_h):
    @functools.partial(
        pl.kernel,
        out_type=jax.ShapeDtypeStruct((NC, NPAD), jnp.float32),
        mesh=_mesh(),
        scratch_types=[
            pltpu.VMEM((NWIN, WN), jnp.int32),
            pltpu.VMEM((WN,), jnp.float32),
            pltpu.VMEM((128,), jnp.float32),
            pltpu.VMEM_SHARED((NPAD,), jnp.float32),
            pltpu.SemaphoreType.DMA,
        ],
    )
    def k(dst_h, ones_hbm, z_hbm, deg_out, idx_v, ones_v, z_v, acc, sem):
        ci = lax.axis_index("c")
        si = lax.axis_index("s")
        w = ci * NS + si
        pltpu.sync_copy(dst_h.at[w], idx_v)
        pltpu.sync_copy(ones_hbm, ones_v)
        pltpu.sync_copy(z_hbm, z_v)

        @pl.loop(0, RW // 128)
        def _zero(j):
            pltpu.sync_copy(z_v, acc.at[pl.ds(si * RW + j * 128, 128)])

        plsc.subcore_barrier()

        @pl.loop(0, NWIN)
        def _scat(j):
            pltpu.async_copy(ones_v, acc.at[idx_v.at[j]], sem, add=True)

        @pl.loop(0, NWIN)
        def _drain(j):
            pltpu.make_async_copy(ones_v, acc.at[pl.ds(0, WN)], sem).wait()

        plsc.subcore_barrier()

        @pl.loop(0, RW // 128)
        def _dump(j):
            off = si * RW + j * 128
            pltpu.sync_copy(acc.at[pl.ds(off, 128)], deg_out.at[ci, pl.ds(off, 128)])

    return k(dst3, ones_h, zrow_h)


def _sc_agg(us, src3, dst3, zt_h):
    C = len(us)
    outs = tuple(jax.ShapeDtypeStruct((NC, NPAD, 128), jnp.float32) for _ in range(C))

    @functools.partial(
        pl.kernel,
        out_type=outs,
        mesh=_mesh(),
        scratch_types=[
            pltpu.VMEM((NWIN, WN), jnp.int32),
            pltpu.VMEM((NWIN, WN), jnp.int32),
            pltpu.VMEM((WN, 128), jnp.float32),
            pltpu.VMEM((WN, 128), jnp.float32),
            pltpu.VMEM((WN, 128), jnp.float32),
            pltpu.VMEM_SHARED((NPAD, 128), jnp.float32),
            pltpu.SemaphoreType.DMA,
            pltpu.SemaphoreType.DMA,
            pltpu.SemaphoreType.DMA,
            pltpu.SemaphoreType.DMA,
            pltpu.SemaphoreType.DMA,
            pltpu.SemaphoreType.DMA,
        ],
    )
    def k(*refs):
        u_refs = refs[:C]
        src_h, dst_h, zt = refs[C], refs[C + 1], refs[C + 2]
        out_refs = refs[C + 3:2 * C + 3]
        sc = refs[2 * C + 3:]
        si_v, di_v = sc[0], sc[1]
        bufs = sc[2:5]
        acc = sc[5]
        gs = sc[6:9]
        ss = sc[9:12]
        ci = lax.axis_index("c")
        si = lax.axis_index("s")
        w = ci * NS + si
        pltpu.sync_copy(src_h.at[w], si_v)
        pltpu.sync_copy(dst_h.at[w], di_v)
        for c in range(C):
            u = u_refs[c]
            out = out_refs[c]

            pltpu.sync_copy(zt, bufs[2])
            for p in range(RW // WN):
                pltpu.async_copy(
                    bufs[2], acc.at[pl.ds(si * RW + p * WN, WN)], gs[1])
            for p in range(RW // WN):
                pltpu.make_async_copy(
                    bufs[2], acc.at[pl.ds(0, WN)], gs[1]).wait()

            plsc.subcore_barrier()

            for t in range(2):
                pltpu.async_copy(u.at[si_v.at[t]], bufs[t], gs[t])

            @pl.loop(0, NWIN - 2, step=3)
            def _edges(j, u=u):
                for t in range(3):
                    tn = (t + 2) % 3
                    pltpu.make_async_copy(u.at[pl.ds(0, WN)], bufs[t], gs[t]).wait()
                    pltpu.async_copy(bufs[t], acc.at[di_v.at[j + t]], ss[t], add=True)

                    @pl.when(j + t > 0)
                    def _ws(tn=tn):
                        pltpu.make_async_copy(
                            bufs[tn], acc.at[pl.ds(0, WN)], ss[tn]).wait()

                    pltpu.async_copy(u.at[si_v.at[j + t + 2]], bufs[tn], gs[tn])

            pltpu.make_async_copy(u.at[pl.ds(0, WN)], bufs[0], gs[0]).wait()
            pltpu.async_copy(bufs[0], acc.at[di_v.at[NWIN - 2]], ss[0], add=True)
            pltpu.make_async_copy(bufs[2], acc.at[pl.ds(0, WN)], ss[2]).wait()
            pltpu.make_async_copy(u.at[pl.ds(0, WN)], bufs[1], gs[1]).wait()
            pltpu.async_copy(bufs[1], acc.at[di_v.at[NWIN - 1]], ss[1], add=True)
            pltpu.make_async_copy(bufs[0], acc.at[pl.ds(0, WN)], ss[0]).wait()
            pltpu.make_async_copy(bufs[1], acc.at[pl.ds(0, WN)], ss[1]).wait()
            plsc.subcore_barrier()

            for p in range(RW // 128):
                off = si * RW + p * 128
                pltpu.async_copy(
                    acc.at[pl.ds(off, 128)], out.at[ci, pl.ds(off, 128)], gs[0])
            for p in range(RW // 128):
                pltpu.make_async_copy(
                    acc.at[pl.ds(0, 128)], out.at[ci, pl.ds(0, 128)], gs[0]).wait()

            plsc.subcore_barrier()

    return k(*us, src3, dst3, zt_h)


def _tc_dinv(deg2, xp, C):
    def body(d_ref, x_ref, dv_ref, *outs):
        d = d_ref[...]
        t = d[0] + d[1] + 1.0
        r = lax.rsqrt(jnp.maximum(t, 1.0))
        dv = jnp.broadcast_to(r[:, None], (NPAD, 128))
        dv_ref[...] = dv
        xx = x_ref[...]
        for c in range(C):
            outs[c][...] = xx[:, c * 128:(c + 1) * 128] * dv

    res = pl.pallas_call(
        body,
        grid=(1,),
        in_specs=[
            pl.BlockSpec((NC, NPAD), lambda i: (0, 0)),
            pl.BlockSpec((NPAD, C * 128), lambda i: (0, 0)),
        ],
        out_specs=(
            pl.BlockSpec((NPAD, 128), lambda i: (0, 0)),
            *[pl.BlockSpec((NPAD, 128), lambda i: (0, 0)) for _ in range(C)],
        ),
        out_shape=(
            jax.ShapeDtypeStruct((NPAD, 128), jnp.float32),
            *[jax.ShapeDtypeStruct((NPAD, 128), jnp.float32) for _ in range(C)],
        ),
    )(deg2, xp)
    return res[0], tuple(res[1:])


def _tc_fuse_mm(aggs, us, dinv, W, brow, dout):
    C = len(us)

    def body(*refs):
        a = refs[:C]
        u = refs[C:2 * C]
        dv = refs[2 * C][...]
        w_all = refs[2 * C + 1][...]
        b_ = refs[2 * C + 2][...]
        o_ref = refs[2 * C + 3]
        st_ref = refs[2 * C + 4]
        m = pl.program_id(0)
        s = jnp.concatenate(
            [(a[c][...][0] + a[c][...][1] + u[c][...]) * dv for c in range(C)],
            axis=1)
        o = jnp.broadcast_to(b_, (BM, dout)) + _dot3(s, w_all)
        o_ref[...] = o
        rows = lax.broadcasted_iota(jnp.int32, (BM, 1), 0) + m * BM
        om = jnp.where(rows < N, o, 0.0)
        part = jnp.concatenate(
            [jnp.sum(om, axis=0)[None, :], jnp.sum(om * om, axis=0)[None, :],
             jnp.zeros((6, dout), jnp.float32)], axis=0)

        @pl.when(m == 0)
        def _init():
            st_ref[...] = part

        @pl.when(m > 0)
        def _acc():
            st_ref[...] += part

    din = C * 128
    return pl.pallas_call(
        body,
        grid=(NPAD // BM,),
        in_specs=(
            [pl.BlockSpec((NC, BM, 128), lambda m: (0, m, 0)) for _ in range(C)]
            + [pl.BlockSpec((BM, 128), lambda m: (m, 0)) for _ in range(C)]
            + [
                pl.BlockSpec((BM, 128), lambda m: (m, 0)),
                pl.BlockSpec((din, dout), lambda m: (0, 0)),
                pl.BlockSpec((1, dout), lambda m: (0, 0)),
            ]
        ),
        out_specs=(
            pl.BlockSpec((BM, dout), lambda m: (m, 0)),
            pl.BlockSpec((8, dout), lambda m: (0, 0)),
        ),
        out_shape=(
            jax.ShapeDtypeStruct((NPAD, dout), jnp.float32),
            jax.ShapeDtypeStruct((8, dout), jnp.float32),
        ),
    )(*aggs, *us, dinv, W, brow)


def _tc_mm(us, W, dout):
    C = len(us)

    def body(*refs):
        u = refs[:C]
        w_all = refs[C][...]
        o_ref = refs[C + 1]
        s = jnp.concatenate([u[c][...] for c in range(C)], axis=1)
        o_ref[...] = _dot3(s, w_all)

    din = C * 128
    return pl.pallas_call(
        body,
        grid=(NPAD // BM,),
        in_specs=(
            [pl.BlockSpec((BM, 128), lambda m: (m, 0)) for _ in range(C)]
            + [pl.BlockSpec((din, dout), lambda m: (0, 0))]
        ),
        out_specs=pl.BlockSpec((BM, dout), lambda m: (m, 0)),
        out_shape=jax.ShapeDtypeStruct((NPAD, dout), jnp.float32),
    )(*us, W)


def _tc_bnapply(o, st, grow, berow, dinv, C, d):
    def body(o_ref, st_ref, g_ref, be_ref, dv_ref, *outs):
        st_ = st_ref[...]
        mu = st_[0] / N
        var = st_[1] / N - mu * mu
        scale = lax.rsqrt(var + 1e-5) * g_ref[...][0]
        xb = (o_ref[...] - mu) * scale + be_ref[...][0]
        xb = jnp.maximum(xb, 0.0)
        dv = dv_ref[...]
        for c in range(C):
            outs[c][...] = xb[:, c * 128:(c + 1) * 128] * dv

    return pl.pallas_call(
        body,
        grid=(NPAD // BM,),
        in_specs=[
            pl.BlockSpec((BM, d), lambda m: (m, 0)),
            pl.BlockSpec((8, d), lambda m: (0, 0)),
            pl.BlockSpec((1, d), lambda m: (0, 0)),
            pl.BlockSpec((1, d), lambda m: (0, 0)),
            pl.BlockSpec((BM, 128), lambda m: (m, 0)),
        ],
        out_specs=tuple(pl.BlockSpec((BM, 128), lambda m: (m, 0)) for _ in range(C)),
        out_shape=tuple(jax.ShapeDtypeStruct((NPAD, 128), jnp.float32) for _ in range(C)),
    )(o, st, grow, berow, dinv)


def _tc_o3_stats(a3, v, dinv, b3row):
    def body(a_ref, v_ref, dv_ref, b_ref, o_ref, st_ref):
        a = a_ref[...]
        o = (a[0] + a[1] + v_ref[...]) * dv_ref[...] + b_ref[...][0]
        o_ref[...] = o
        s1 = jnp.sum(o, axis=0)
        s2 = jnp.sum(o * o, axis=0)
        st_ref[...] = jnp.concatenate(
            [s1[None, :], s2[None, :], jnp.zeros((6, 128), jnp.float32)], axis=0)

    return pl.pallas_call(
        body,
        grid=(1,),
        in_specs=[
            pl.BlockSpec((NC, N, 128), lambda i: (0, 0, 0)),
            pl.BlockSpec((N, 128), lambda i: (0, 0)),
            pl.BlockSpec((N, 128), lambda i: (0, 0)),
            pl.BlockSpec((1, 128), lambda i: (0, 0)),
        ],
        out_specs=(
            pl.BlockSpec((N, 128), lambda i: (0, 0)),
            pl.BlockSpec((8, 128), lambda i: (0, 0)),
        ),
        out_shape=(
            jax.ShapeDtypeStruct((N, 128), jnp.float32),
            jax.ShapeDtypeStruct((8, 128), jnp.float32),
        ),
    )(a3, v, dinv, b3row)


def _tc_pool(o3, st3, g3row, be3row, batch_row):
    def body(o_ref, st_ref, g_ref, be_ref, b_ref, out_ref):
        st_ = st_ref[...]
        mu = st_[0] / N
        var = st_[1] / N - mu * mu
        scale = lax.rsqrt(var + 1e-5) * g_ref[...][0]
        xb = (o_ref[...] - mu) * scale + be_ref[...][0]
        seg = b_ref[...]
        gid = lax.broadcasted_iota(jnp.int32, (G, 1), 0)
        oht = (seg == gid).astype(jnp.float32)
        sums = _dot3(oht, xb)
        cnt = jnp.sum(oht, axis=1)[:, None]
        pooled = sums / jnp.maximum(cnt, 1.0)
        nrm = jnp.sqrt(jnp.sum(pooled * pooled, axis=1, keepdims=True))
        out_ref[...] = pooled / jnp.maximum(nrm, 1e-12)

    return pl.pallas_call(
        body,
        grid=(1,),
        in_specs=[
            pl.BlockSpec((N, 128), lambda i: (0, 0)),
            pl.BlockSpec((8, 128), lambda i: (0, 0)),
            pl.BlockSpec((1, 128), lambda i: (0, 0)),
            pl.BlockSpec((1, 128), lambda i: (0, 0)),
            pl.BlockSpec((1, N), lambda i: (0, 0)),
        ],
        out_specs=pl.BlockSpec((G, 128), lambda i: (0, 0)),
        out_shape=jax.ShapeDtypeStruct((G, 128), jnp.float32),
    )(o3, st3, g3row, be3row, batch_row)


def kernel(x, edge_index, batch, W1, b1, g1, be1, W2, b2, g2, be2, W3, b3, g3, be3):
    f32 = jnp.float32
    src = edge_index[0]
    dst = edge_index[1]
    pad = (N + jnp.arange(EPAD - E, dtype=jnp.int32) % (NPAD - N)).astype(jnp.int32)
    src3 = jnp.concatenate([src, pad]).reshape(NW, NWIN, WN)
    dst3 = jnp.concatenate([dst, pad]).reshape(NW, NWIN, WN)
    xp = jnp.pad(x, ((0, NPAD - N), (0, 0)))
    ones_h = jnp.ones((WN,), f32)
    zrow_h = jnp.zeros((128,), f32)
    zt_h = jnp.zeros((WN, 128), f32)
    b1r, b2r, b3r = b1.reshape(1, -1), b2.reshape(1, -1), b3.reshape(1, -1)
    g1r, g2r, g3r = g1.reshape(1, -1), g2.reshape(1, -1), g3.reshape(1, -1)
    be1r, be2r, be3r = be1.reshape(1, -1), be2.reshape(1, -1), be3.reshape(1, -1)
    batch_row = batch.reshape(1, N)

    deg2 = _sc_degree(dst3, ones_h, zrow_h)
    dinv, u1 = _tc_dinv(deg2, xp, 2)
    a1 = _sc_agg(u1, src3, dst3, zt_h)
    o1, st1 = _tc_fuse_mm(a1, u1, dinv, W1, b1r, 512)
    u2 = _tc_bnapply(o1, st1, g1r, be1r, dinv, 4, 512)

    a2 = _sc_agg(u2, src3, dst3, zt_h)
    o2, st2 = _tc_fuse_mm(a2, u2, dinv, W2, b2r, 512)
    q = _tc_bnapply(o2, st2, g2r, be2r, dinv, 4, 512)

    v = _tc_mm(q, W3, 128)
    a3 = _sc_agg((v,), src3, dst3, zt_h)
    o3, st3 = _tc_o3_stats(a3[0], v, dinv, b3r)
    return _tc_pool(o3, st3, g3r, be3r, batch_row)


# speedup vs baseline: 1.0272x; 1.0094x over previous
import functools

import jax
import jax.numpy as jnp
from jax import lax
from jax.experimental import pallas as pl
from jax.experimental.pallas import tpu as pltpu
from jax.experimental.pallas import tpu_sc as plsc

N = 10000
E = 160000
G = 256
NPAD = 10240
EPAD = 163840
NC, NS = 2, 16
NW = NC * NS
EW = EPAD // NW
WN = 64
NWIN = EW // WN
RW = NPAD // NS
BM = 1024


def _dot3(s, w):
    dn = (((1,), (0,)), ((), ()))
    sh = s.astype(jnp.bfloat16)
    sl = (s - sh.astype(jnp.float32)).astype(jnp.bfloat16)
    wh = w.astype(jnp.bfloat16)
    wl = (w - wh.astype(jnp.float32)).astype(jnp.bfloat16)
    out = lax.dot_general(sh, wh, dn, preferred_element_type=jnp.float32)
    out = out + lax.dot_general(sh, wl, dn, preferred_element_type=jnp.float32)
    out = out + lax.dot_general(sl, wh, dn, preferred_element_type=jnp.float32)
    return out


def _mesh():
    return plsc.VectorSubcoreMesh(core_axis_name="c", subcore_axis_name="s")


def _sc_degree(dst3, ones_h, zrow_h):
    @functools.partial(
        pl.kernel,
        out_type=jax.ShapeDtypeStruct((NC, NPAD), jnp.float32),
        mesh=_mesh(),
        scratch_types=[
            pltpu.VMEM((NWIN, WN), jnp.int32),
            pltpu.VMEM((WN,), jnp.float32),
            pltpu.VMEM((128,), jnp.float32),
            pltpu.VMEM_SHARED((NPAD,), jnp.float32),
            pltpu.SemaphoreType.DMA,
        ],
    )
    def k(dst_h, ones_hbm, z_hbm, deg_out, idx_v, ones_v, z_v, acc, sem):
        ci = lax.axis_index("c")
        si = lax.axis_index("s")
        w = ci * NS + si
        pltpu.sync_copy(dst_h.at[w], idx_v)
        pltpu.sync_copy(ones_hbm, ones_v)
        pltpu.sync_copy(z_hbm, z_v)

        @pl.loop(0, RW // 128)
        def _zero(j):
            pltpu.sync_copy(z_v, acc.at[pl.ds(si * RW + j * 128, 128)])

        plsc.subcore_barrier()

        @pl.loop(0, NWIN)
        def _scat(j):
            pltpu.async_copy(ones_v, acc.at[idx_v.at[j]], sem, add=True)

        @pl.loop(0, NWIN)
        def _drain(j):
            pltpu.make_async_copy(ones_v, acc.at[pl.ds(0, WN)], sem).wait()

        plsc.subcore_barrier()

        @pl.loop(0, RW // 128)
        def _dump(j):
            off = si * RW + j * 128
            pltpu.sync_copy(acc.at[pl.ds(off, 128)], deg_out.at[ci, pl.ds(off, 128)])

    return k(dst3, ones_h, zrow_h)


def _sc_agg(us, src3, dst3, zt_h):
    C = len(us)
    outs = tuple(jax.ShapeDtypeStruct((NC, NPAD, 128), jnp.float32) for _ in range(C))

    @functools.partial(
        pl.kernel,
        out_type=outs,
        mesh=_mesh(),
        scratch_types=[
            pltpu.VMEM((NWIN, WN), jnp.int32),
            pltpu.VMEM((NWIN, WN), jnp.int32),
            pltpu.VMEM((WN, 128), jnp.float32),
            pltpu.VMEM((WN, 128), jnp.float32),
            pltpu.VMEM((WN, 128), jnp.float32),
            pltpu.VMEM_SHARED((NPAD, 128), jnp.float32),
            pltpu.SemaphoreType.DMA,
            pltpu.SemaphoreType.DMA,
            pltpu.SemaphoreType.DMA,
            pltpu.SemaphoreType.DMA,
            pltpu.SemaphoreType.DMA,
            pltpu.SemaphoreType.DMA,
        ],
    )
    def k(*refs):
        u_refs = refs[:C]
        src_h, dst_h, zt = refs[C], refs[C + 1], refs[C + 2]
        out_refs = refs[C + 3:2 * C + 3]
        sc = refs[2 * C + 3:]
        si_v, di_v = sc[0], sc[1]
        bufs = sc[2:5]
        acc = sc[5]
        gs = sc[6:9]
        ss = sc[9:12]
        ci = lax.axis_index("c")
        si = lax.axis_index("s")
        w = ci * NS + si
        pltpu.sync_copy(src_h.at[w], si_v)
        pltpu.sync_copy(dst_h.at[w], di_v)
        for c in range(C):
            u = u_refs[c]
            out = out_refs[c]

            pltpu.sync_copy(zt, bufs[2])
            for p in range(RW // WN):
                pltpu.async_copy(
                    bufs[2], acc.at[pl.ds(si * RW + p * WN, WN)], gs[1])
            for p in range(RW // WN):
                pltpu.make_async_copy(
                    bufs[2], acc.at[pl.ds(0, WN)], gs[1]).wait()

            plsc.subcore_barrier()

            for t in range(2):
                pltpu.async_copy(u.at[si_v.at[t]], bufs[t], gs[t])

            @pl.loop(0, NWIN - 2, step=3)
            def _edges(j, u=u):
                for t in range(3):
                    tn = (t + 2) % 3
                    pltpu.make_async_copy(u.at[pl.ds(0, WN)], bufs[t], gs[t]).wait()
                    pltpu.async_copy(bufs[t], acc.at[di_v.at[j + t]], ss[t], add=True)

                    @pl.when(j + t > 0)
                    def _ws(tn=tn):
                        pltpu.make_async_copy(
                            bufs[tn], acc.at[pl.ds(0, WN)], ss[tn]).wait()

                    pltpu.async_copy(u.at[si_v.at[j + t + 2]], bufs[tn], gs[tn])

            pltpu.make_async_copy(u.at[pl.ds(0, WN)], bufs[0], gs[0]).wait()
            pltpu.async_copy(bufs[0], acc.at[di_v.at[NWIN - 2]], ss[0], add=True)
            pltpu.make_async_copy(bufs[2], acc.at[pl.ds(0, WN)], ss[2]).wait()
            pltpu.make_async_copy(u.at[pl.ds(0, WN)], bufs[1], gs[1]).wait()
            pltpu.async_copy(bufs[1], acc.at[di_v.at[NWIN - 1]], ss[1], add=True)
            pltpu.make_async_copy(bufs[0], acc.at[pl.ds(0, WN)], ss[0]).wait()
            pltpu.make_async_copy(bufs[1], acc.at[pl.ds(0, WN)], ss[1]).wait()
            plsc.subcore_barrier()

            for p in range(RW // 128):
                off = si * RW + p * 128
                pltpu.async_copy(
                    acc.at[pl.ds(off, 128)], out.at[ci, pl.ds(off, 128)], gs[0])
            for p in range(RW // 128):
                pltpu.make_async_copy(
                    acc.at[pl.ds(0, 128)], out.at[ci, pl.ds(0, 128)], gs[0]).wait()

            plsc.subcore_barrier()

    return k(*us, src3, dst3, zt_h)


def _tc_dinv(deg2, xp, C):
    def body(d_ref, x_ref, dv_ref, *outs):
        d = d_ref[...]
        t = d[0] + d[1] + 1.0
        r = lax.rsqrt(jnp.maximum(t, 1.0))
        dv = jnp.broadcast_to(r[:, None], (NPAD, 128))
        dv_ref[...] = dv
        xx = x_ref[...]
        for c in range(C):
            outs[c][...] = xx[:, c * 128:(c + 1) * 128] * dv

    res = pl.pallas_call(
        body,
        grid=(1,),
        in_specs=[
            pl.BlockSpec((NC, NPAD), lambda i: (0, 0)),
            pl.BlockSpec((NPAD, C * 128), lambda i: (0, 0)),
        ],
        out_specs=(
            pl.BlockSpec((NPAD, 128), lambda i: (0, 0)),
            *[pl.BlockSpec((NPAD, 128), lambda i: (0, 0)) for _ in range(C)],
        ),
        out_shape=(
            jax.ShapeDtypeStruct((NPAD, 128), jnp.float32),
            *[jax.ShapeDtypeStruct((NPAD, 128), jnp.float32) for _ in range(C)],
        ),
    )(deg2, xp)
    return res[0], tuple(res[1:])


def _tc_fuse_mm(aggs, us, dinv, W, brow, dout):
    C = len(us)

    def body(*refs):
        a = refs[:C]
        u = refs[C:2 * C]
        dv = refs[2 * C][...]
        w_all = refs[2 * C + 1][...]
        b_ = refs[2 * C + 2][...]
        o_ref = refs[2 * C + 3]
        st_ref = refs[2 * C + 4]
        m = pl.program_id(0)
        s = jnp.concatenate(
            [(a[c][...][0] + a[c][...][1] + u[c][...]) * dv for c in range(C)],
            axis=1)
        o = jnp.broadcast_to(b_, (BM, dout)) + _dot3(s, w_all)
        o_ref[...] = o
        rows = lax.broadcasted_iota(jnp.int32, (BM, 1), 0) + m * BM
        om = jnp.where(rows < N, o, 0.0)
        part = jnp.concatenate(
            [jnp.sum(om, axis=0)[None, :], jnp.sum(om * om, axis=0)[None, :],
             jnp.zeros((6, dout), jnp.float32)], axis=0)

        @pl.when(m == 0)
        def _init():
            st_ref[...] = part

        @pl.when(m > 0)
        def _acc():
            st_ref[...] += part

    din = C * 128
    return pl.pallas_call(
        body,
        grid=(NPAD // BM,),
        in_specs=(
            [pl.BlockSpec((NC, BM, 128), lambda m: (0, m, 0)) for _ in range(C)]
            + [pl.BlockSpec((BM, 128), lambda m: (m, 0)) for _ in range(C)]
            + [
                pl.BlockSpec((BM, 128), lambda m: (m, 0)),
                pl.BlockSpec((din, dout), lambda m: (0, 0)),
                pl.BlockSpec((1, dout), lambda m: (0, 0)),
            ]
        ),
        out_specs=(
            pl.BlockSpec((BM, dout), lambda m: (m, 0)),
            pl.BlockSpec((8, dout), lambda m: (0, 0)),
        ),
        out_shape=(
            jax.ShapeDtypeStruct((NPAD, dout), jnp.float32),
            jax.ShapeDtypeStruct((8, dout), jnp.float32),
        ),
    )(*aggs, *us, dinv, W, brow)


def _tc_mm(us, W, dout):
    C = len(us)

    def body(*refs):
        u = refs[:C]
        w_all = refs[C][...]
        o_ref = refs[C + 1]
        s = jnp.concatenate([u[c][...] for c in range(C)], axis=1)
        o_ref[...] = _dot3(s, w_all)

    din = C * 128
    return pl.pallas_call(
        body,
        grid=(NPAD // BM,),
        in_specs=(
            [pl.BlockSpec((BM, 128), lambda m: (m, 0)) for _ in range(C)]
            + [pl.BlockSpec((din, dout), lambda m: (0, 0))]
        ),
        out_specs=pl.BlockSpec((BM, dout), lambda m: (m, 0)),
        out_shape=jax.ShapeDtypeStruct((NPAD, dout), jnp.float32),
    )(*us, W)


def _tc_bnapply(o, st, grow, berow, dinv, C, d):
    def body(o_ref, st_ref, g_ref, be_ref, dv_ref, *outs):
        st_ = st_ref[...]
        mu = st_[0] / N
        var = st_[1] / N - mu * mu
        scale = lax.rsqrt(var + 1e-5) * g_ref[...][0]
        xb = (o_ref[...] - mu) * scale + be_ref[...][0]
        xb = jnp.maximum(xb, 0.0)
        dv = dv_ref[...]
        for c in range(C):
            outs[c][...] = xb[:, c * 128:(c + 1) * 128] * dv

    return pl.pallas_call(
        body,
        grid=(NPAD // BM,),
        in_specs=[
            pl.BlockSpec((BM, d), lambda m: (m, 0)),
            pl.BlockSpec((8, d), lambda m: (0, 0)),
            pl.BlockSpec((1, d), lambda m: (0, 0)),
            pl.BlockSpec((1, d), lambda m: (0, 0)),
            pl.BlockSpec((BM, 128), lambda m: (m, 0)),
        ],
        out_specs=tuple(pl.BlockSpec((BM, 128), lambda m: (m, 0)) for _ in range(C)),
        out_shape=tuple(jax.ShapeDtypeStruct((NPAD, 128), jnp.float32) for _ in range(C)),
    )(o, st, grow, berow, dinv)


def _tc_tail(a3, v, dinv, b3row, g3row, be3row, batch_row):
    def body(a_ref, v_ref, dv_ref, b_ref, g_ref, be_ref, bt_ref, out_ref):
        a = a_ref[...]
        o = (a[0] + a[1] + v_ref[...]) * dv_ref[...] + b_ref[...][0]
        s1 = jnp.sum(o, axis=0)
        s2 = jnp.sum(o * o, axis=0)
        mu = s1 / N
        var = s2 / N - mu * mu
        scale = lax.rsqrt(var + 1e-5) * g_ref[...][0]
        xb = (o - mu) * scale + be_ref[...][0]
        seg = bt_ref[...]
        gid = lax.broadcasted_iota(jnp.int32, (G, 1), 0)
        oht = (seg == gid).astype(jnp.float32)
        sums = _dot3(oht, xb)
        cnt = jnp.sum(oht, axis=1)[:, None]
        pooled = sums / jnp.maximum(cnt, 1.0)
        nrm = jnp.sqrt(jnp.sum(pooled * pooled, axis=1, keepdims=True))
        out_ref[...] = pooled / jnp.maximum(nrm, 1e-12)

    return pl.pallas_call(
        body,
        grid=(1,),
        in_specs=[
            pl.BlockSpec((NC, N, 128), lambda i: (0, 0, 0)),
            pl.BlockSpec((N, 128), lambda i: (0, 0)),
            pl.BlockSpec((N, 128), lambda i: (0, 0)),
            pl.BlockSpec((1, 128), lambda i: (0, 0)),
            pl.BlockSpec((1, 128), lambda i: (0, 0)),
            pl.BlockSpec((1, 128), lambda i: (0, 0)),
            pl.BlockSpec((1, N), lambda i: (0, 0)),
        ],
        out_specs=pl.BlockSpec((G, 128), lambda i: (0, 0)),
        out_shape=jax.ShapeDtypeStruct((G, 128), jnp.float32),
    )(a3, v, dinv, b3row, g3row, be3row, batch_row)


def kernel(x, edge_index, batch, W1, b1, g1, be1, W2, b2, g2, be2, W3, b3, g3, be3):
    f32 = jnp.float32
    src = edge_index[0]
    dst = edge_index[1]
    pad = (N + jnp.arange(EPAD - E, dtype=jnp.int32) % (NPAD - N)).astype(jnp.int32)
    src3 = jnp.concatenate([src, pad]).reshape(NW, NWIN, WN)
    dst3 = jnp.concatenate([dst, pad]).reshape(NW, NWIN, WN)
    xp = jnp.pad(x, ((0, NPAD - N), (0, 0)))
    ones_h = jnp.ones((WN,), f32)
    zrow_h = jnp.zeros((128,), f32)
    zt_h = jnp.zeros((WN, 128), f32)
    b1r, b2r, b3r = b1.reshape(1, -1), b2.reshape(1, -1), b3.reshape(1, -1)
    g1r, g2r, g3r = g1.reshape(1, -1), g2.reshape(1, -1), g3.reshape(1, -1)
    be1r, be2r, be3r = be1.reshape(1, -1), be2.reshape(1, -1), be3.reshape(1, -1)
    batch_row = batch.reshape(1, N)

    deg2 = _sc_degree(dst3, ones_h, zrow_h)
    dinv, u1 = _tc_dinv(deg2, xp, 2)
    a1 = _sc_agg(u1, src3, dst3, zt_h)
    o1, st1 = _tc_fuse_mm(a1, u1, dinv, W1, b1r, 512)
    u2 = _tc_bnapply(o1, st1, g1r, be1r, dinv, 4, 512)

    a2 = _sc_agg(u2, src3, dst3, zt_h)
    o2, st2 = _tc_fuse_mm(a2, u2, dinv, W2, b2r, 512)
    q = _tc_bnapply(o2, st2, g2r, be2r, dinv, 4, 512)

    v = _tc_mm(q, W3, 128)
    a3 = _sc_agg((v,), src3, dst3, zt_h)
    return _tc_tail(a3[0], v, dinv, b3r, g3r, be3r, batch_row)
